# Initial kernel scaffold; baseline (speedup 1.0000x reference)
import jax
import jax.numpy as jnp
from jax.experimental import pallas as pl


def kernel(label):
    raise NotImplementedError("write your pallas kernel here")


# speedup vs baseline: 26.6638x; 26.6638x over previous
import jax
import jax.numpy as jnp
from jax.experimental import pallas as pl

_C = 150
_H = 512
_W = 512
_CB = 10


def _onehot_body(lab_ref, out_ref):
    c0 = pl.program_id(0) * _CB
    cls = c0 + jax.lax.broadcasted_iota(jnp.int32, (_CB, _H, _W), 0)
    out_ref[...] = (lab_ref[...][None, :, :] == cls).astype(jnp.float32)


def kernel(label):
    return pl.pallas_call(
        _onehot_body,
        grid=(_C // _CB,),
        in_specs=[pl.BlockSpec((_H, _W), lambda i: (0, 0))],
        out_specs=pl.BlockSpec((_CB, _H, _W), lambda i: (i, 0, 0)),
        out_shape=jax.ShapeDtypeStruct((_C, _H, _W), jnp.float32),
    )(label)
